# Initial kernel scaffold; baseline (speedup 1.0000x reference)
import jax
import jax.numpy as jnp
from jax.experimental import pallas as pl


def kernel(users, pos_items, neg_items, user_emb, item_emb, W_gc_0, b_gc_0, W_bi_0, b_bi_0, W_gc_1, b_gc_1, W_bi_1, b_bi_1, W_gc_2, b_gc_2, W_bi_2, b_bi_2, user_b, prod_b, alpha, global_bias, adj_rows, adj_cols, adj_vals):
    raise NotImplementedError("write your pallas kernel here")


# speedup vs baseline: 3.7726x; 3.7726x over previous
import functools

import jax
import jax.numpy as jnp
from jax import lax
from jax.experimental import pallas as pl
from jax.experimental.pallas import tpu as pltpu
from jax.experimental.pallas import tpu_sc as plsc

N_USER = 25000
N_ITEM = 25000
N = N_USER + N_ITEM
NP = 50176
D = 64
DH = D // 2
E = 800000
B = 4096

NC = 2
NS = 16
NW = NC * NS

EP = E // NS
CHUNK = 80
NCHUNK = EP // CHUNK
SB = 25
NSB = NCHUNK // SB
RPT = NP // NS

BPT = B // NW

_f32 = jnp.float32
_i32 = jnp.int32

_sc_mesh = plsc.VectorSubcoreMesh(core_axis_name="c", subcore_axis_name="s")


@functools.partial(
    pl.kernel,
    out_type=(
        jax.ShapeDtypeStruct((NP, DH), _f32),
        jax.ShapeDtypeStruct((NP, DH), _f32),
    ),
    mesh=_sc_mesh,
    compiler_params=pltpu.CompilerParams(use_tc_tiling_on_sc=False),
    scratch_types=[
        pltpu.VMEM_SHARED((NP, DH), _f32),
        pltpu.VMEM((SB, CHUNK), _i32),
        pltpu.VMEM((SB, CHUNK), _i32),
        pltpu.VMEM((SB, CHUNK), _f32),
        pltpu.VMEM((CHUNK, DH), _f32),
    ],
)
def _spmm(ego_l, ego_r, rows_hbm, cols_hbm, vals_hbm, zeros_hbm,
          side_l, side_r, acc, cols_v, rows_v, vals_v, gbuf):
    c = lax.axis_index("c")
    s = lax.axis_index("s")
    r0 = s * RPT

    pltpu.sync_copy(zeros_hbm.at[pl.ds(r0, RPT)], acc.at[pl.ds(r0, RPT)])
    plsc.subcore_barrier()

    def superblock(b, carry):
        pltpu.sync_copy(cols_hbm.at[s, b], cols_v)
        pltpu.sync_copy(rows_hbm.at[s, b], rows_v)
        pltpu.sync_copy(vals_hbm.at[s, b], vals_v)

        def chunk(j, carry2):
            cidx = cols_v.at[j]

            @pl.when(c == 0)
            def _():
                pltpu.sync_copy(ego_l.at[cidx], gbuf)

            @pl.when(c == 1)
            def _():
                pltpu.sync_copy(ego_r.at[cidx], gbuf)

            def scale16(g, carry3):
                vvec = vals_v[j, pl.ds(g * 16, 16)]
                i0 = g * 16
                for k in range(16):
                    i = i0 + k
                    v = vvec[k]
                    gbuf[i, pl.ds(0, 16)] = gbuf[i, pl.ds(0, 16)] * v
                    gbuf[i, pl.ds(16, 16)] = gbuf[i, pl.ds(16, 16)] * v
                return carry3

            lax.fori_loop(0, CHUNK // 16, scale16, 0)
            pltpu.sync_copy(gbuf, acc.at[rows_v.at[j]], add=True)
            return carry2

        lax.fori_loop(0, SB, chunk, 0)
        return carry

    lax.fori_loop(0, NSB, superblock, 0)
    plsc.subcore_barrier()

    @pl.when(c == 0)
    def _():
        pltpu.sync_copy(acc.at[pl.ds(r0, RPT)], side_l.at[pl.ds(r0, RPT)])

    @pl.when(c == 1)
    def _():
        pltpu.sync_copy(acc.at[pl.ds(r0, RPT)], side_r.at[pl.ds(r0, RPT)])


BN = 512


def _dense_body(side_l, side_r, ego_l, ego_r, wgc, bgc, wbi, bbi,
                out_l, out_r, norm_out):
    sde = jnp.concatenate([side_l[...], side_r[...]], axis=1)
    ego = jnp.concatenate([ego_l[...], ego_r[...]], axis=1)
    y = jnp.dot(sde, wgc[...], preferred_element_type=_f32) + bgc[...]
    y = y + jnp.dot(ego * sde, wbi[...] + bbi[...], preferred_element_type=_f32)
    y = jnp.where(y >= 0, y, 0.2 * y)
    out_l[...] = y[:, :DH]
    out_r[...] = y[:, DH:]
    nrm = jnp.maximum(jnp.sqrt(jnp.sum(y * y, axis=1, keepdims=True)), 1e-12)
    norm_out[...] = y / nrm


_dense = pl.pallas_call(
    _dense_body,
    grid=(NP // BN,),
    in_specs=[
        pl.BlockSpec((BN, DH), lambda i: (i, 0)),
        pl.BlockSpec((BN, DH), lambda i: (i, 0)),
        pl.BlockSpec((BN, DH), lambda i: (i, 0)),
        pl.BlockSpec((BN, DH), lambda i: (i, 0)),
        pl.BlockSpec((D, D), lambda i: (0, 0)),
        pl.BlockSpec((1, D), lambda i: (0, 0)),
        pl.BlockSpec((D, D), lambda i: (0, 0)),
        pl.BlockSpec((1, D), lambda i: (0, 0)),
    ],
    out_specs=[
        pl.BlockSpec((BN, DH), lambda i: (i, 0)),
        pl.BlockSpec((BN, DH), lambda i: (i, 0)),
        pl.BlockSpec((BN, D), lambda i: (i, 0)),
    ],
    out_shape=[
        jax.ShapeDtypeStruct((NP, DH), _f32),
        jax.ShapeDtypeStruct((NP, DH), _f32),
        jax.ShapeDtypeStruct((NP, D), _f32),
    ],
)


DA = 4 * D


@functools.partial(
    pl.kernel,
    out_type=(
        jax.ShapeDtypeStruct((B, DA), _f32),
        jax.ShapeDtypeStruct((B, DA), _f32),
        jax.ShapeDtypeStruct((B, 16), _f32),
        jax.ShapeDtypeStruct((B, 16), _f32),
    ),
    mesh=_sc_mesh,
    compiler_params=pltpu.CompilerParams(use_tc_tiling_on_sc=False),
    scratch_types=[
        pltpu.VMEM((BPT,), _i32),
        pltpu.VMEM((BPT,), _i32),
        pltpu.VMEM((BPT,), _i32),
        pltpu.VMEM((BPT, DA), _f32),
        pltpu.VMEM((BPT, 16), _f32),
    ],
)
def _gather(all_e, idxu_hbm, idxi_hbm, idxit_hbm, ubt_hbm, pbt_hbm,
            u_out, i_out, ub_out, pb_out,
            idxu_v, idxi_v, idxit_v, rbuf, bbuf):
    c = lax.axis_index("c")
    s = lax.axis_index("s")
    w = s * NC + c
    base = w * BPT

    pltpu.sync_copy(idxu_hbm.at[w], idxu_v)
    pltpu.sync_copy(idxi_hbm.at[w], idxi_v)
    pltpu.sync_copy(idxit_hbm.at[w], idxit_v)

    pltpu.sync_copy(all_e.at[idxu_v], rbuf)
    pltpu.sync_copy(rbuf, u_out.at[pl.ds(base, BPT)])
    pltpu.sync_copy(all_e.at[idxi_v], rbuf)
    pltpu.sync_copy(rbuf, i_out.at[pl.ds(base, BPT)])

    pltpu.sync_copy(ubt_hbm.at[idxu_v], bbuf)
    pltpu.sync_copy(bbuf, ub_out.at[pl.ds(base, BPT)])
    pltpu.sync_copy(pbt_hbm.at[idxit_v], bbuf)
    pltpu.sync_copy(bbuf, pb_out.at[pl.ds(base, BPT)])


BF = 1024


def _final_body(u, i, ub, pb, al, gb, lo, pr):
    d = jnp.sum(u[...] * i[...], axis=1, keepdims=True)
    alv = al[...]
    gbv = gb[...]
    lg = alv[0, 0] * d + ub[...][:, :1] + pb[...][:, :1] + gbv[0, 0]
    lo[...] = lg
    pr[...] = jax.nn.sigmoid(lg)


_final = pl.pallas_call(
    _final_body,
    grid=(B // BF,),
    in_specs=[
        pl.BlockSpec((BF, DA), lambda i: (i, 0)),
        pl.BlockSpec((BF, DA), lambda i: (i, 0)),
        pl.BlockSpec((BF, 16), lambda i: (i, 0)),
        pl.BlockSpec((BF, 16), lambda i: (i, 0)),
        pl.BlockSpec((1, 1), lambda i: (0, 0)),
        pl.BlockSpec((1, 1), lambda i: (0, 0)),
    ],
    out_specs=[
        pl.BlockSpec((BF, 1), lambda i: (i, 0)),
        pl.BlockSpec((BF, 1), lambda i: (i, 0)),
    ],
    out_shape=[
        jax.ShapeDtypeStruct((B, 1), _f32),
        jax.ShapeDtypeStruct((B, 1), _f32),
    ],
)


def kernel(users, pos_items, neg_items, user_emb, item_emb,
           W_gc_0, b_gc_0, W_bi_0, b_bi_0,
           W_gc_1, b_gc_1, W_bi_1, b_bi_1,
           W_gc_2, b_gc_2, W_bi_2, b_bi_2,
           user_b, prod_b, alpha, global_bias,
           adj_rows, adj_cols, adj_vals):
    ego0 = jnp.concatenate([user_emb, item_emb], axis=0)
    ego0 = jnp.pad(ego0, ((0, NP - N), (0, 0)))
    ego_l = ego0[:, :DH]
    ego_r = ego0[:, DH:]

    rows_r = adj_rows.reshape(NS, NSB, SB, CHUNK)
    cols_r = adj_cols.reshape(NS, NSB, SB, CHUNK)
    vals_r = adj_vals.reshape(NS, NSB, SB, CHUNK)
    zeros = jnp.zeros((NP, DH), _f32)

    layer_params = [(W_gc_0, b_gc_0, W_bi_0, b_bi_0),
                    (W_gc_1, b_gc_1, W_bi_1, b_bi_1),
                    (W_gc_2, b_gc_2, W_bi_2, b_bi_2)]

    norm_parts = [ego0]
    for (wgc, bgc, wbi, bbi) in layer_params:
        side_l, side_r = _spmm(ego_l, ego_r, rows_r, cols_r, vals_r, zeros)
        ego_l, ego_r, norm_e = _dense(side_l, side_r, ego_l, ego_r,
                                      wgc, bgc, wbi, bbi)
        norm_parts.append(norm_e)

    all_e = jnp.concatenate(norm_parts, axis=1)

    items_idx = pos_items + neg_items
    idxu = users.reshape(NW, BPT)
    idxi = (N_USER + items_idx).reshape(NW, BPT)
    idxit = items_idx.reshape(NW, BPT)
    ubt = jnp.broadcast_to(user_b[:, None], (N_USER, 16))
    pbt = jnp.broadcast_to(prod_b[:, None], (N_ITEM, 16))

    u_g, i_g, ub_g, pb_g = _gather(all_e, idxu, idxi, idxit, ubt, pbt)

    al = alpha.reshape(1, 1)
    gb = global_bias.reshape(1, 1)
    logits, prediction = _final(u_g, i_g, ub_g, pb_g, al, gb)
    return (u_g, i_g, logits, prediction)


# speedup vs baseline: 6.5938x; 1.7478x over previous
import functools

import jax
import jax.numpy as jnp
from jax import lax
from jax.experimental import pallas as pl
from jax.experimental.pallas import tpu as pltpu
from jax.experimental.pallas import tpu_sc as plsc

N_USER = 25000
N_ITEM = 25000
N = N_USER + N_ITEM
NP = 50176
D = 64
DH = D // 2
E = 800000
B = 4096

NC = 2
NS = 16
NW = NC * NS

EP = E // NS
CHUNK = 80
NCHUNK = EP // CHUNK
SB = 25
NSB = NCHUNK // SB
RPT = NP // NS
RING = 5
LOOK = 4

BPT = B // NW

_f32 = jnp.float32
_i32 = jnp.int32

_sc_mesh = plsc.VectorSubcoreMesh(core_axis_name="c", subcore_axis_name="s")


@functools.partial(
    pl.kernel,
    out_type=(
        jax.ShapeDtypeStruct((NP, DH), _f32),
        jax.ShapeDtypeStruct((NP, DH), _f32),
    ),
    mesh=_sc_mesh,
    compiler_params=pltpu.CompilerParams(use_tc_tiling_on_sc=False),
    scratch_types=[
        pltpu.VMEM_SHARED((NP, DH), _f32),
        pltpu.VMEM((SB, CHUNK), _i32),
        pltpu.VMEM((SB, CHUNK), _i32),
        pltpu.VMEM((SB, CHUNK), _f32),
        pltpu.VMEM((RING, CHUNK, DH), _f32),
        pltpu.SemaphoreType.DMA((RING,)),
        pltpu.SemaphoreType.DMA((RING,)),
    ],
)
def _spmm(ego_l, ego_r, rows_hbm, cols_hbm, vals_hbm, zeros_hbm,
          side_l, side_r, acc, cols_v, rows_v, vals_v, gbufs, gsem, ssem):
    c = lax.axis_index("c")
    s = lax.axis_index("s")
    r0 = s * RPT

    pltpu.sync_copy(zeros_hbm.at[pl.ds(r0, RPT)], acc.at[pl.ds(r0, RPT)])
    plsc.subcore_barrier()

    def superblock(b, carry):
        pltpu.sync_copy(cols_hbm.at[s, b], cols_v)
        pltpu.sync_copy(rows_hbm.at[s, b], rows_v)
        pltpu.sync_copy(vals_hbm.at[s, b], vals_v)

        gds = [None] * RING
        sds = [None] * RING

        def fire_gather(j):
            r = j % RING
            cidx = cols_v.at[j]
            gbuf = gbufs.at[r]

            @pl.when(c == 0)
            def _():
                pltpu.async_copy(ego_l.at[cidx], gbuf, gsem.at[r])

            @pl.when(c == 1)
            def _():
                pltpu.async_copy(ego_r.at[cidx], gbuf, gsem.at[r])

            gds[r] = pltpu.make_async_copy(ego_l.at[cidx], gbuf, gsem.at[r])

        def process(j):
            r = j % RING
            gds[r].wait()
            gbuf = gbufs.at[r]

            def scale16(g, carry3, _j=j, _gbuf=gbuf):
                vvec = vals_v[_j, pl.ds(g * 16, 16)]
                i0 = g * 16
                for k in range(16):
                    i = i0 + k
                    v = vvec[k]
                    _gbuf[i, pl.ds(0, 16)] = _gbuf[i, pl.ds(0, 16)] * v
                    _gbuf[i, pl.ds(16, 16)] = _gbuf[i, pl.ds(16, 16)] * v
                return carry3

            lax.fori_loop(0, CHUNK // 16, scale16, 0)
            sds[r] = pltpu.async_copy(gbuf, acc.at[rows_v.at[j]],
                                      ssem.at[r], add=True)

        for j in range(SB + LOOK):
            if j < SB:
                r = j % RING
                if sds[r] is not None:
                    sds[r].wait()
                fire_gather(j)
            if j >= LOOK:
                process(j - LOOK)

        for r in range(RING):
            if sds[r] is not None:
                sds[r].wait()
        return carry

    lax.fori_loop(0, NSB, superblock, 0)
    plsc.subcore_barrier()

    @pl.when(c == 0)
    def _():
        pltpu.sync_copy(acc.at[pl.ds(r0, RPT)], side_l.at[pl.ds(r0, RPT)])

    @pl.when(c == 1)
    def _():
        pltpu.sync_copy(acc.at[pl.ds(r0, RPT)], side_r.at[pl.ds(r0, RPT)])


BN = 512


def _dense_body(side_l, side_r, ego_l, ego_r, wgc, bgc, wbi, bbi,
                out_l, out_r, norm_out):
    sde = jnp.concatenate([side_l[...], side_r[...]], axis=1)
    ego = jnp.concatenate([ego_l[...], ego_r[...]], axis=1)
    y = jnp.dot(sde, wgc[...], preferred_element_type=_f32) + bgc[...]
    y = y + jnp.dot(ego * sde, wbi[...] + bbi[...], preferred_element_type=_f32)
    y = jnp.where(y >= 0, y, 0.2 * y)
    out_l[...] = y[:, :DH]
    out_r[...] = y[:, DH:]
    nrm = jnp.maximum(jnp.sqrt(jnp.sum(y * y, axis=1, keepdims=True)), 1e-12)
    norm_out[...] = y / nrm


_dense = pl.pallas_call(
    _dense_body,
    grid=(NP // BN,),
    in_specs=[
        pl.BlockSpec((BN, DH), lambda i: (i, 0)),
        pl.BlockSpec((BN, DH), lambda i: (i, 0)),
        pl.BlockSpec((BN, DH), lambda i: (i, 0)),
        pl.BlockSpec((BN, DH), lambda i: (i, 0)),
        pl.BlockSpec((D, D), lambda i: (0, 0)),
        pl.BlockSpec((1, D), lambda i: (0, 0)),
        pl.BlockSpec((D, D), lambda i: (0, 0)),
        pl.BlockSpec((1, D), lambda i: (0, 0)),
    ],
    out_specs=[
        pl.BlockSpec((BN, DH), lambda i: (i, 0)),
        pl.BlockSpec((BN, DH), lambda i: (i, 0)),
        pl.BlockSpec((BN, D), lambda i: (i, 0)),
    ],
    out_shape=[
        jax.ShapeDtypeStruct((NP, DH), _f32),
        jax.ShapeDtypeStruct((NP, DH), _f32),
        jax.ShapeDtypeStruct((NP, D), _f32),
    ],
)


DA = 4 * D


@functools.partial(
    pl.kernel,
    out_type=(
        jax.ShapeDtypeStruct((B, DA), _f32),
        jax.ShapeDtypeStruct((B, DA), _f32),
        jax.ShapeDtypeStruct((B, 16), _f32),
        jax.ShapeDtypeStruct((B, 16), _f32),
    ),
    mesh=_sc_mesh,
    compiler_params=pltpu.CompilerParams(use_tc_tiling_on_sc=False),
    scratch_types=[
        pltpu.VMEM((BPT,), _i32),
        pltpu.VMEM((BPT,), _i32),
        pltpu.VMEM((BPT,), _i32),
        pltpu.VMEM((BPT, DA), _f32),
        pltpu.VMEM((BPT, 16), _f32),
    ],
)
def _gather(all_e, idxu_hbm, idxi_hbm, idxit_hbm, ubt_hbm, pbt_hbm,
            u_out, i_out, ub_out, pb_out,
            idxu_v, idxi_v, idxit_v, rbuf, bbuf):
    c = lax.axis_index("c")
    s = lax.axis_index("s")
    w = s * NC + c
    base = w * BPT

    pltpu.sync_copy(idxu_hbm.at[w], idxu_v)
    pltpu.sync_copy(idxi_hbm.at[w], idxi_v)
    pltpu.sync_copy(idxit_hbm.at[w], idxit_v)

    pltpu.sync_copy(all_e.at[idxu_v], rbuf)
    pltpu.sync_copy(rbuf, u_out.at[pl.ds(base, BPT)])
    pltpu.sync_copy(all_e.at[idxi_v], rbuf)
    pltpu.sync_copy(rbuf, i_out.at[pl.ds(base, BPT)])

    pltpu.sync_copy(ubt_hbm.at[idxu_v], bbuf)
    pltpu.sync_copy(bbuf, ub_out.at[pl.ds(base, BPT)])
    pltpu.sync_copy(pbt_hbm.at[idxit_v], bbuf)
    pltpu.sync_copy(bbuf, pb_out.at[pl.ds(base, BPT)])


BF = 1024


def _final_body(u, i, ub, pb, al, gb, lo, pr):
    d = jnp.sum(u[...] * i[...], axis=1, keepdims=True)
    alv = al[...]
    gbv = gb[...]
    lg = alv[0, 0] * d + ub[...][:, :1] + pb[...][:, :1] + gbv[0, 0]
    lo[...] = lg
    pr[...] = jax.nn.sigmoid(lg)


_final = pl.pallas_call(
    _final_body,
    grid=(B // BF,),
    in_specs=[
        pl.BlockSpec((BF, DA), lambda i: (i, 0)),
        pl.BlockSpec((BF, DA), lambda i: (i, 0)),
        pl.BlockSpec((BF, 16), lambda i: (i, 0)),
        pl.BlockSpec((BF, 16), lambda i: (i, 0)),
        pl.BlockSpec((1, 1), lambda i: (0, 0)),
        pl.BlockSpec((1, 1), lambda i: (0, 0)),
    ],
    out_specs=[
        pl.BlockSpec((BF, 1), lambda i: (i, 0)),
        pl.BlockSpec((BF, 1), lambda i: (i, 0)),
    ],
    out_shape=[
        jax.ShapeDtypeStruct((B, 1), _f32),
        jax.ShapeDtypeStruct((B, 1), _f32),
    ],
)


def kernel(users, pos_items, neg_items, user_emb, item_emb,
           W_gc_0, b_gc_0, W_bi_0, b_bi_0,
           W_gc_1, b_gc_1, W_bi_1, b_bi_1,
           W_gc_2, b_gc_2, W_bi_2, b_bi_2,
           user_b, prod_b, alpha, global_bias,
           adj_rows, adj_cols, adj_vals):
    ego0 = jnp.concatenate([user_emb, item_emb], axis=0)
    ego0 = jnp.pad(ego0, ((0, NP - N), (0, 0)))
    ego_l = ego0[:, :DH]
    ego_r = ego0[:, DH:]

    rows_r = adj_rows.reshape(NS, NSB, SB, CHUNK)
    cols_r = adj_cols.reshape(NS, NSB, SB, CHUNK)
    vals_r = adj_vals.reshape(NS, NSB, SB, CHUNK)
    zeros = jnp.zeros((NP, DH), _f32)

    layer_params = [(W_gc_0, b_gc_0, W_bi_0, b_bi_0),
                    (W_gc_1, b_gc_1, W_bi_1, b_bi_1),
                    (W_gc_2, b_gc_2, W_bi_2, b_bi_2)]

    norm_parts = [ego0]
    for (wgc, bgc, wbi, bbi) in layer_params:
        side_l, side_r = _spmm(ego_l, ego_r, rows_r, cols_r, vals_r, zeros)
        ego_l, ego_r, norm_e = _dense(side_l, side_r, ego_l, ego_r,
                                      wgc, bgc, wbi, bbi)
        norm_parts.append(norm_e)

    all_e = jnp.concatenate(norm_parts, axis=1)

    items_idx = pos_items + neg_items
    idxu = users.reshape(NW, BPT)
    idxi = (N_USER + items_idx).reshape(NW, BPT)
    idxit = items_idx.reshape(NW, BPT)
    ubt = jnp.broadcast_to(user_b[:, None], (N_USER, 16))
    pbt = jnp.broadcast_to(prod_b[:, None], (N_ITEM, 16))

    u_g, i_g, ub_g, pb_g = _gather(all_e, idxu, idxi, idxit, ubt, pbt)

    al = alpha.reshape(1, 1)
    gb = global_bias.reshape(1, 1)
    logits, prediction = _final(u_g, i_g, ub_g, pb_g, al, gb)
    return (u_g, i_g, logits, prediction)


# speedup vs baseline: 6.8572x; 1.0400x over previous
import functools

import jax
import jax.numpy as jnp
from jax import lax
from jax.experimental import pallas as pl
from jax.experimental.pallas import tpu as pltpu
from jax.experimental.pallas import tpu_sc as plsc

N_USER = 25000
N_ITEM = 25000
N = N_USER + N_ITEM
NP = 50176
D = 64
DH = D // 2
E = 800000
B = 4096

NC = 2
NS = 16
NW = NC * NS

EP = E // NS
CHUNK = 128
SB = 28
NSB = 14
EPP = CHUNK * SB * NSB
PADE = EPP - EP
RPT = NP // NS
RING = 4
LOOK = 3

BPT = B // NW

_f32 = jnp.float32
_i32 = jnp.int32

_sc_mesh = plsc.VectorSubcoreMesh(core_axis_name="c", subcore_axis_name="s")


@functools.partial(
    pl.kernel,
    out_type=(
        jax.ShapeDtypeStruct((NP, DH), _f32),
        jax.ShapeDtypeStruct((NP, DH), _f32),
    ),
    mesh=_sc_mesh,
    compiler_params=pltpu.CompilerParams(use_tc_tiling_on_sc=False),
    scratch_types=[
        pltpu.VMEM_SHARED((NP, DH), _f32),
        pltpu.VMEM((SB, CHUNK), _i32),
        pltpu.VMEM((SB, CHUNK), _i32),
        pltpu.VMEM((SB, CHUNK), _f32),
        pltpu.VMEM((RING, CHUNK, DH), _f32),
        pltpu.SemaphoreType.DMA((RING,)),
        pltpu.SemaphoreType.DMA((RING,)),
    ],
)
def _spmm(ego_l, ego_r, rows_hbm, cols_hbm, vals_hbm, zeros_hbm,
          side_l, side_r, acc, cols_v, rows_v, vals_v, gbufs, gsem, ssem):
    c = lax.axis_index("c")
    s = lax.axis_index("s")
    r0 = s * RPT

    pltpu.sync_copy(zeros_hbm.at[pl.ds(r0, RPT)], acc.at[pl.ds(r0, RPT)])
    plsc.subcore_barrier()

    def superblock(b, carry):
        pltpu.sync_copy(cols_hbm.at[s, b], cols_v)
        pltpu.sync_copy(rows_hbm.at[s, b], rows_v)
        pltpu.sync_copy(vals_hbm.at[s, b], vals_v)

        gds = [None] * RING
        sds = [None] * RING

        def fire_gather(j):
            r = j % RING
            cidx = cols_v.at[j]
            gbuf = gbufs.at[r]

            @pl.when(c == 0)
            def _():
                pltpu.async_copy(ego_l.at[cidx], gbuf, gsem.at[r])

            @pl.when(c == 1)
            def _():
                pltpu.async_copy(ego_r.at[cidx], gbuf, gsem.at[r])

            gds[r] = pltpu.make_async_copy(ego_l.at[cidx], gbuf, gsem.at[r])

        def process(j):
            r = j % RING
            gds[r].wait()
            gbuf = gbufs.at[r]

            def scale16(g, carry3, _j=j, _gbuf=gbuf):
                vvec = vals_v[_j, pl.ds(g * 16, 16)]
                i0 = g * 16
                for k in range(16):
                    i = i0 + k
                    v = vvec[k]
                    _gbuf[i, pl.ds(0, 16)] = _gbuf[i, pl.ds(0, 16)] * v
                    _gbuf[i, pl.ds(16, 16)] = _gbuf[i, pl.ds(16, 16)] * v
                return carry3

            lax.fori_loop(0, CHUNK // 16, scale16, 0)
            sds[r] = pltpu.async_copy(gbuf, acc.at[rows_v.at[j]],
                                      ssem.at[r], add=True)

        for j in range(SB + LOOK):
            if j < SB:
                r = j % RING
                if sds[r] is not None:
                    sds[r].wait()
                fire_gather(j)
            if j >= LOOK:
                process(j - LOOK)

        for r in range(RING):
            if sds[r] is not None:
                sds[r].wait()
        return carry

    lax.fori_loop(0, NSB, superblock, 0)
    plsc.subcore_barrier()

    @pl.when(c == 0)
    def _():
        pltpu.sync_copy(acc.at[pl.ds(r0, RPT)], side_l.at[pl.ds(r0, RPT)])

    @pl.when(c == 1)
    def _():
        pltpu.sync_copy(acc.at[pl.ds(r0, RPT)], side_r.at[pl.ds(r0, RPT)])


BN = 512


def _dense_body(side_l, side_r, ego_l, ego_r, wgc, bgc, wbi, bbi,
                out_l, out_r, norm_out):
    sde = jnp.concatenate([side_l[...], side_r[...]], axis=1)
    ego = jnp.concatenate([ego_l[...], ego_r[...]], axis=1)
    y = jnp.dot(sde, wgc[...], preferred_element_type=_f32) + bgc[...]
    y = y + jnp.dot(ego * sde, wbi[...] + bbi[...], preferred_element_type=_f32)
    y = jnp.where(y >= 0, y, 0.2 * y)
    out_l[...] = y[:, :DH]
    out_r[...] = y[:, DH:]
    nrm = jnp.maximum(jnp.sqrt(jnp.sum(y * y, axis=1, keepdims=True)), 1e-12)
    norm_out[...] = y / nrm


_dense = pl.pallas_call(
    _dense_body,
    grid=(NP // BN,),
    in_specs=[
        pl.BlockSpec((BN, DH), lambda i: (i, 0)),
        pl.BlockSpec((BN, DH), lambda i: (i, 0)),
        pl.BlockSpec((BN, DH), lambda i: (i, 0)),
        pl.BlockSpec((BN, DH), lambda i: (i, 0)),
        pl.BlockSpec((D, D), lambda i: (0, 0)),
        pl.BlockSpec((1, D), lambda i: (0, 0)),
        pl.BlockSpec((D, D), lambda i: (0, 0)),
        pl.BlockSpec((1, D), lambda i: (0, 0)),
    ],
    out_specs=[
        pl.BlockSpec((BN, DH), lambda i: (i, 0)),
        pl.BlockSpec((BN, DH), lambda i: (i, 0)),
        pl.BlockSpec((BN, D), lambda i: (i, 0)),
    ],
    out_shape=[
        jax.ShapeDtypeStruct((NP, DH), _f32),
        jax.ShapeDtypeStruct((NP, DH), _f32),
        jax.ShapeDtypeStruct((NP, D), _f32),
    ],
)


DA = 4 * D


@functools.partial(
    pl.kernel,
    out_type=(
        jax.ShapeDtypeStruct((B, DA), _f32),
        jax.ShapeDtypeStruct((B, DA), _f32),
        jax.ShapeDtypeStruct((B, 16), _f32),
        jax.ShapeDtypeStruct((B, 16), _f32),
    ),
    mesh=_sc_mesh,
    compiler_params=pltpu.CompilerParams(use_tc_tiling_on_sc=False),
    scratch_types=[
        pltpu.VMEM((BPT,), _i32),
        pltpu.VMEM((BPT,), _i32),
        pltpu.VMEM((BPT,), _i32),
        pltpu.VMEM((BPT, DA), _f32),
        pltpu.VMEM((BPT, 16), _f32),
    ],
)
def _gather(all_e, idxu_hbm, idxi_hbm, idxit_hbm, ubt_hbm, pbt_hbm,
            u_out, i_out, ub_out, pb_out,
            idxu_v, idxi_v, idxit_v, rbuf, bbuf):
    c = lax.axis_index("c")
    s = lax.axis_index("s")
    w = s * NC + c
    base = w * BPT

    pltpu.sync_copy(idxu_hbm.at[w], idxu_v)
    pltpu.sync_copy(idxi_hbm.at[w], idxi_v)
    pltpu.sync_copy(idxit_hbm.at[w], idxit_v)

    pltpu.sync_copy(all_e.at[idxu_v], rbuf)
    pltpu.sync_copy(rbuf, u_out.at[pl.ds(base, BPT)])
    pltpu.sync_copy(all_e.at[idxi_v], rbuf)
    pltpu.sync_copy(rbuf, i_out.at[pl.ds(base, BPT)])

    pltpu.sync_copy(ubt_hbm.at[idxu_v], bbuf)
    pltpu.sync_copy(bbuf, ub_out.at[pl.ds(base, BPT)])
    pltpu.sync_copy(pbt_hbm.at[idxit_v], bbuf)
    pltpu.sync_copy(bbuf, pb_out.at[pl.ds(base, BPT)])


BF = 1024


def _final_body(u, i, ub, pb, al, gb, lo, pr):
    d = jnp.sum(u[...] * i[...], axis=1, keepdims=True)
    alv = al[...]
    gbv = gb[...]
    lg = alv[0, 0] * d + ub[...][:, :1] + pb[...][:, :1] + gbv[0, 0]
    lo[...] = lg
    pr[...] = jax.nn.sigmoid(lg)


_final = pl.pallas_call(
    _final_body,
    grid=(B // BF,),
    in_specs=[
        pl.BlockSpec((BF, DA), lambda i: (i, 0)),
        pl.BlockSpec((BF, DA), lambda i: (i, 0)),
        pl.BlockSpec((BF, 16), lambda i: (i, 0)),
        pl.BlockSpec((BF, 16), lambda i: (i, 0)),
        pl.BlockSpec((1, 1), lambda i: (0, 0)),
        pl.BlockSpec((1, 1), lambda i: (0, 0)),
    ],
    out_specs=[
        pl.BlockSpec((BF, 1), lambda i: (i, 0)),
        pl.BlockSpec((BF, 1), lambda i: (i, 0)),
    ],
    out_shape=[
        jax.ShapeDtypeStruct((B, 1), _f32),
        jax.ShapeDtypeStruct((B, 1), _f32),
    ],
)


def kernel(users, pos_items, neg_items, user_emb, item_emb,
           W_gc_0, b_gc_0, W_bi_0, b_bi_0,
           W_gc_1, b_gc_1, W_bi_1, b_bi_1,
           W_gc_2, b_gc_2, W_bi_2, b_bi_2,
           user_b, prod_b, alpha, global_bias,
           adj_rows, adj_cols, adj_vals):
    ego0 = jnp.concatenate([user_emb, item_emb], axis=0)
    ego0 = jnp.pad(ego0, ((0, NP - N), (0, 0)))
    ego_l = ego0[:, :DH]
    ego_r = ego0[:, DH:]

    rows_r = jnp.concatenate(
        [adj_rows.reshape(NS, EP), jnp.full((NS, PADE), N, _i32)],
        axis=1).reshape(NS, NSB, SB, CHUNK)
    cols_r = jnp.concatenate(
        [adj_cols.reshape(NS, EP), jnp.zeros((NS, PADE), _i32)],
        axis=1).reshape(NS, NSB, SB, CHUNK)
    vals_r = jnp.concatenate(
        [adj_vals.reshape(NS, EP), jnp.zeros((NS, PADE), _f32)],
        axis=1).reshape(NS, NSB, SB, CHUNK)
    zeros = jnp.zeros((NP, DH), _f32)

    layer_params = [(W_gc_0, b_gc_0, W_bi_0, b_bi_0),
                    (W_gc_1, b_gc_1, W_bi_1, b_bi_1),
                    (W_gc_2, b_gc_2, W_bi_2, b_bi_2)]

    norm_parts = [ego0]
    for (wgc, bgc, wbi, bbi) in layer_params:
        side_l, side_r = _spmm(ego_l, ego_r, rows_r, cols_r, vals_r, zeros)
        ego_l, ego_r, norm_e = _dense(side_l, side_r, ego_l, ego_r,
                                      wgc, bgc, wbi, bbi)
        norm_parts.append(norm_e)

    all_e = jnp.concatenate(norm_parts, axis=1)

    items_idx = pos_items + neg_items
    idxu = users.reshape(NW, BPT)
    idxi = (N_USER + items_idx).reshape(NW, BPT)
    idxit = items_idx.reshape(NW, BPT)
    ubt = jnp.broadcast_to(user_b[:, None], (N_USER, 16))
    pbt = jnp.broadcast_to(prod_b[:, None], (N_ITEM, 16))

    u_g, i_g, ub_g, pb_g = _gather(all_e, idxu, idxi, idxit, ubt, pbt)

    al = alpha.reshape(1, 1)
    gb = global_bias.reshape(1, 1)
    logits, prediction = _final(u_g, i_g, ub_g, pb_g, al, gb)
    return (u_g, i_g, logits, prediction)


# speedup vs baseline: 7.2215x; 1.0531x over previous
import functools

import jax
import jax.numpy as jnp
from jax import lax
from jax.experimental import pallas as pl
from jax.experimental.pallas import tpu as pltpu
from jax.experimental.pallas import tpu_sc as plsc

N_USER = 25000
N_ITEM = 25000
N = N_USER + N_ITEM
NP = 50176
D = 64
DH = D // 2
E = 800000
B = 4096

NC = 2
NS = 16
NW = NC * NS

EP = E // NS
CHUNK = 128
SB = 28
NSB = 14
EPP = CHUNK * SB * NSB
PADE = EPP - EP
RPT = NP // NS
RING = 4
LOOK = 3

BPT = B // NW

_f32 = jnp.float32
_i32 = jnp.int32

_sc_mesh = plsc.VectorSubcoreMesh(core_axis_name="c", subcore_axis_name="s")


@functools.partial(
    pl.kernel,
    out_type=(
        jax.ShapeDtypeStruct((NP, DH), _f32),
        jax.ShapeDtypeStruct((NP, DH), _f32),
    ),
    mesh=_sc_mesh,
    compiler_params=pltpu.CompilerParams(use_tc_tiling_on_sc=False),
    scratch_types=[
        pltpu.VMEM_SHARED((NP, DH), _f32),
        pltpu.VMEM((SB, CHUNK), _i32),
        pltpu.VMEM((SB, CHUNK), _i32),
        pltpu.VMEM((SB, CHUNK), _f32),
        pltpu.VMEM((RING, CHUNK, DH), _f32),
        pltpu.SemaphoreType.DMA((RING,)),
        pltpu.SemaphoreType.DMA((RING,)),
    ],
)
def _spmm(ego_l, ego_r, rows_hbm, cols_hbm, vals_hbm, zeros_hbm,
          side_l, side_r, acc, cols_v, rows_v, vals_v, gbufs, gsem, ssem):
    c = lax.axis_index("c")
    s = lax.axis_index("s")
    r0 = s * RPT

    pltpu.sync_copy(zeros_hbm.at[pl.ds(r0, RPT)], acc.at[pl.ds(r0, RPT)])
    plsc.subcore_barrier()

    def superblock(b, carry):
        pltpu.sync_copy(cols_hbm.at[s, b], cols_v)
        pltpu.sync_copy(rows_hbm.at[s, b], rows_v)
        pltpu.sync_copy(vals_hbm.at[s, b], vals_v)

        gds = [None] * RING
        sds = [None] * RING

        def fire_gather(j):
            r = j % RING
            cidx = cols_v.at[j]
            gbuf = gbufs.at[r]

            @pl.when(c == 0)
            def _():
                pltpu.async_copy(ego_l.at[cidx], gbuf, gsem.at[r])

            @pl.when(c == 1)
            def _():
                pltpu.async_copy(ego_r.at[cidx], gbuf, gsem.at[r])

            gds[r] = pltpu.make_async_copy(ego_l.at[cidx], gbuf, gsem.at[r])

        def process(j):
            r = j % RING
            gds[r].wait()
            gbuf = gbufs.at[r]

            def scale16(g, carry3, _j=j, _gbuf=gbuf):
                vvec = vals_v[_j, pl.ds(g * 16, 16)]
                i0 = g * 16
                for k in range(16):
                    i = i0 + k
                    v = vvec[k]
                    _gbuf[i, pl.ds(0, 16)] = _gbuf[i, pl.ds(0, 16)] * v
                    _gbuf[i, pl.ds(16, 16)] = _gbuf[i, pl.ds(16, 16)] * v
                return carry3

            lax.fori_loop(0, CHUNK // 16, scale16, 0)
            sds[r] = pltpu.async_copy(gbuf, acc.at[rows_v.at[j]],
                                      ssem.at[r], add=True)

        for j in range(SB + LOOK):
            if j < SB:
                r = j % RING
                if sds[r] is not None:
                    sds[r].wait()
                fire_gather(j)
            if j >= LOOK:
                process(j - LOOK)

        for r in range(RING):
            if sds[r] is not None:
                sds[r].wait()
        return carry

    lax.fori_loop(0, NSB, superblock, 0)
    plsc.subcore_barrier()

    @pl.when(c == 0)
    def _():
        pltpu.sync_copy(acc.at[pl.ds(r0, RPT)], side_l.at[pl.ds(r0, RPT)])

    @pl.when(c == 1)
    def _():
        pltpu.sync_copy(acc.at[pl.ds(r0, RPT)], side_r.at[pl.ds(r0, RPT)])


BN = 512


def _dense_body(side_l, side_r, ego_l, ego_r, wgc, bgc, wbi, bbi,
                out_l, out_r, norm_out):
    sde = jnp.concatenate([side_l[...], side_r[...]], axis=1)
    ego = jnp.concatenate([ego_l[...], ego_r[...]], axis=1)
    y = jnp.dot(sde, wgc[...], preferred_element_type=_f32) + bgc[...]
    y = y + jnp.dot(ego * sde, wbi[...] + bbi[...], preferred_element_type=_f32)
    y = jnp.where(y >= 0, y, 0.2 * y)
    out_l[...] = y[:, :DH]
    out_r[...] = y[:, DH:]
    nrm = jnp.maximum(jnp.sqrt(jnp.sum(y * y, axis=1, keepdims=True)), 1e-12)
    norm_out[...] = y / nrm


_dense = pl.pallas_call(
    _dense_body,
    grid=(NP // BN,),
    in_specs=[
        pl.BlockSpec((BN, DH), lambda i: (i, 0)),
        pl.BlockSpec((BN, DH), lambda i: (i, 0)),
        pl.BlockSpec((BN, DH), lambda i: (i, 0)),
        pl.BlockSpec((BN, DH), lambda i: (i, 0)),
        pl.BlockSpec((D, D), lambda i: (0, 0)),
        pl.BlockSpec((1, D), lambda i: (0, 0)),
        pl.BlockSpec((D, D), lambda i: (0, 0)),
        pl.BlockSpec((1, D), lambda i: (0, 0)),
    ],
    out_specs=[
        pl.BlockSpec((BN, DH), lambda i: (i, 0)),
        pl.BlockSpec((BN, DH), lambda i: (i, 0)),
        pl.BlockSpec((BN, D), lambda i: (i, 0)),
    ],
    out_shape=[
        jax.ShapeDtypeStruct((NP, DH), _f32),
        jax.ShapeDtypeStruct((NP, DH), _f32),
        jax.ShapeDtypeStruct((NP, D), _f32),
    ],
)


DA = 4 * D


@functools.partial(
    pl.kernel,
    out_type=(
        jax.ShapeDtypeStruct((B, DA), _f32),
        jax.ShapeDtypeStruct((B, DA), _f32),
        jax.ShapeDtypeStruct((B, 16), _f32),
        jax.ShapeDtypeStruct((B, 16), _f32),
    ),
    mesh=_sc_mesh,
    compiler_params=pltpu.CompilerParams(use_tc_tiling_on_sc=False),
    scratch_types=[
        pltpu.VMEM((BPT,), _i32),
        pltpu.VMEM((BPT,), _i32),
        pltpu.VMEM((BPT,), _i32),
        pltpu.VMEM((4, BPT, D), _f32),
        pltpu.VMEM((BPT, 16), _f32),
        pltpu.SemaphoreType.DMA((4,)),
    ],
)
def _gather(t0, t1, t2, t3, idxu_hbm, idxi_hbm, idxit_hbm, ubt_hbm, pbt_hbm,
            u_out, i_out, ub_out, pb_out,
            idxu_v, idxi_v, idxit_v, rbuf, bbuf, rsem):
    c = lax.axis_index("c")
    s = lax.axis_index("s")
    w = s * NC + c
    base = w * BPT

    pltpu.sync_copy(idxu_hbm.at[w], idxu_v)
    pltpu.sync_copy(idxi_hbm.at[w], idxi_v)
    pltpu.sync_copy(idxit_hbm.at[w], idxit_v)

    def gather_to(out_ref, idx_v):
        ds = []
        for k, t in enumerate((t0, t1, t2, t3)):
            ds.append(pltpu.async_copy(t.at[idx_v], rbuf.at[k], rsem.at[k]))
        for k in range(4):
            ds[k].wait()
            pltpu.sync_copy(
                rbuf.at[k],
                out_ref.at[pl.ds(base, BPT), pl.ds(k * D, D)])

    gather_to(u_out, idxu_v)
    gather_to(i_out, idxi_v)

    pltpu.sync_copy(ubt_hbm.at[idxu_v], bbuf)
    pltpu.sync_copy(bbuf, ub_out.at[pl.ds(base, BPT)])
    pltpu.sync_copy(pbt_hbm.at[idxit_v], bbuf)
    pltpu.sync_copy(bbuf, pb_out.at[pl.ds(base, BPT)])


BF = 1024


def _final_body(u, i, ub, pb, al, gb, lo, pr):
    d = jnp.sum(u[...] * i[...], axis=1, keepdims=True)
    alv = al[...]
    gbv = gb[...]
    lg = alv[0, 0] * d + ub[...][:, :1] + pb[...][:, :1] + gbv[0, 0]
    lo[...] = lg
    pr[...] = jax.nn.sigmoid(lg)


_final = pl.pallas_call(
    _final_body,
    grid=(B // BF,),
    in_specs=[
        pl.BlockSpec((BF, DA), lambda i: (i, 0)),
        pl.BlockSpec((BF, DA), lambda i: (i, 0)),
        pl.BlockSpec((BF, 16), lambda i: (i, 0)),
        pl.BlockSpec((BF, 16), lambda i: (i, 0)),
        pl.BlockSpec((1, 1), lambda i: (0, 0)),
        pl.BlockSpec((1, 1), lambda i: (0, 0)),
    ],
    out_specs=[
        pl.BlockSpec((BF, 1), lambda i: (i, 0)),
        pl.BlockSpec((BF, 1), lambda i: (i, 0)),
    ],
    out_shape=[
        jax.ShapeDtypeStruct((B, 1), _f32),
        jax.ShapeDtypeStruct((B, 1), _f32),
    ],
)


def kernel(users, pos_items, neg_items, user_emb, item_emb,
           W_gc_0, b_gc_0, W_bi_0, b_bi_0,
           W_gc_1, b_gc_1, W_bi_1, b_bi_1,
           W_gc_2, b_gc_2, W_bi_2, b_bi_2,
           user_b, prod_b, alpha, global_bias,
           adj_rows, adj_cols, adj_vals):
    ego0 = jnp.concatenate([user_emb, item_emb], axis=0)
    ego0 = jnp.pad(ego0, ((0, NP - N), (0, 0)))
    ego_l = ego0[:, :DH]
    ego_r = ego0[:, DH:]

    rows_r = jnp.concatenate(
        [adj_rows.reshape(NS, EP), jnp.full((NS, PADE), N, _i32)],
        axis=1).reshape(NS, NSB, SB, CHUNK)
    cols_r = jnp.concatenate(
        [adj_cols.reshape(NS, EP), jnp.zeros((NS, PADE), _i32)],
        axis=1).reshape(NS, NSB, SB, CHUNK)
    vals_r = jnp.concatenate(
        [adj_vals.reshape(NS, EP), jnp.zeros((NS, PADE), _f32)],
        axis=1).reshape(NS, NSB, SB, CHUNK)
    zeros = jnp.zeros((NP, DH), _f32)

    layer_params = [(W_gc_0, b_gc_0, W_bi_0, b_bi_0),
                    (W_gc_1, b_gc_1, W_bi_1, b_bi_1),
                    (W_gc_2, b_gc_2, W_bi_2, b_bi_2)]

    norm_parts = [ego0]
    for (wgc, bgc, wbi, bbi) in layer_params:
        side_l, side_r = _spmm(ego_l, ego_r, rows_r, cols_r, vals_r, zeros)
        ego_l, ego_r, norm_e = _dense(side_l, side_r, ego_l, ego_r,
                                      wgc, bgc, wbi, bbi)
        norm_parts.append(norm_e)

    items_idx = pos_items + neg_items
    idxu = users.reshape(NW, BPT)
    idxi = (N_USER + items_idx).reshape(NW, BPT)
    idxit = items_idx.reshape(NW, BPT)
    ubt = jnp.broadcast_to(user_b[:, None], (N_USER, 16))
    pbt = jnp.broadcast_to(prod_b[:, None], (N_ITEM, 16))

    u_g, i_g, ub_g, pb_g = _gather(
        norm_parts[0], norm_parts[1], norm_parts[2], norm_parts[3],
        idxu, idxi, idxit, ubt, pbt)

    al = alpha.reshape(1, 1)
    gb = global_bias.reshape(1, 1)
    logits, prediction = _final(u_g, i_g, ub_g, pb_g, al, gb)
    return (u_g, i_g, logits, prediction)
